# Initial kernel scaffold; baseline (speedup 1.0000x reference)
import jax
import jax.numpy as jnp
from jax.experimental import pallas as pl


def kernel(tracks_x, pvs_x, tt_edge_attr, tp_edge_attr, globals_x, params, tt_edge_index, tp_edge_index):
    raise NotImplementedError("write your pallas kernel here")


# speedup vs baseline: 1.5395x; 1.5395x over previous
import functools

import jax
import jax.numpy as jnp
from jax import lax
from jax.experimental import pallas as pl
from jax.experimental.pallas import tpu as pltpu
from jax.experimental.pallas import tpu_sc as plsc


def _pcall(body, blocked, consts, outs, br):
    n_rows = blocked[0].shape[0]
    grid = (n_rows // br,)
    in_specs = []
    for x in blocked:
        in_specs.append(pl.BlockSpec((br, x.shape[1]), lambda i: (i, 0)))
    for c in consts:
        in_specs.append(pl.BlockSpec(c.shape, lambda i: (0, 0)))
    out_specs, out_shapes = [], []
    for kind, d in outs:
        if kind == 'b':
            out_specs.append(pl.BlockSpec((br, d), lambda i: (i, 0)))
            out_shapes.append(jax.ShapeDtypeStruct((n_rows, d), jnp.float32))
        else:
            out_specs.append(pl.BlockSpec((8, d), lambda i: (0, 0)))
            out_shapes.append(jax.ShapeDtypeStruct((8, d), jnp.float32))
    nb, nc = len(blocked), len(consts)

    def kern(*refs):
        i = pl.program_id(0)
        xs = [refs[k][...] for k in range(nb)]
        cs = [refs[nb + k][...] for k in range(nc)]
        vals = body(i, xs, cs)
        orefs = refs[nb + nc:]
        for (kind, _d), ref, v in zip(outs, orefs, vals):
            if kind == 'b':
                ref[...] = v
            else:
                @pl.when(i == 0)
                def _init(ref=ref, v=v):
                    ref[...] = v

                @pl.when(i != 0)
                def _acc(ref=ref, v=v):
                    ref[...] = ref[...] + v

    return pl.pallas_call(
        kern, grid=grid, in_specs=in_specs, out_specs=out_specs,
        out_shape=out_shapes)(*blocked, *consts)


def _ln_relu(z, g, be):
    m = z.mean(-1, keepdims=True)
    v = jnp.mean((z - m) ** 2, -1, keepdims=True)
    return jax.nn.relu((z - m) * lax.rsqrt(v + 1e-5) * g + be)


def _tail_consts(ps):
    out = [ps[0]['g'].reshape(1, -1), ps[0]['be'].reshape(1, -1)]
    for p in ps[1:]:
        out += [p['W'], p['b'].reshape(1, -1)]
        if 'g' in p:
            out += [p['g'].reshape(1, -1), p['be'].reshape(1, -1)]
    return out


def _tail_chain(z, it, nl):
    h = _ln_relu(z, next(it), next(it))
    for k in range(1, nl):
        h = h @ next(it) + next(it)
        if k < nl - 1:
            h = _ln_relu(h, next(it), next(it))
    return h


def _full_consts(ps):
    out = []
    for p in ps:
        out += [p['W'], p['b'].reshape(1, -1)]
        if 'g' in p:
            out += [p['g'].reshape(1, -1), p['be'].reshape(1, -1)]
    return out


def _full_chain(h, it, nl):
    for k in range(nl):
        h = h @ next(it) + next(it)
        if k < nl - 1:
            h = _ln_relu(h, next(it), next(it))
    return h


def _run_mlp(x, ps, br):
    nl = len(ps)

    def body(i, xs, cs):
        return [_full_chain(xs[0], iter(cs), nl)]

    return _pcall(body, [x], _full_consts(ps),
                  [('b', ps[-1]['W'].shape[1])], br)[0]


_NW = 32
_CHUNK = 128


def _sc_mesh():
    return plsc.VectorSubcoreMesh(core_axis_name="c", subcore_axis_name="s")


def _sc_gather_sum(tab_a, tab_b, idx_a, idx_b, ep):
    dw = tab_a.shape[1]
    pw = ep // _NW
    nch = pw // _CHUNK

    @functools.partial(
        pl.kernel, mesh=_sc_mesh(),
        out_type=jax.ShapeDtypeStruct((ep, dw), jnp.float32),
        scratch_types=[
            pltpu.VMEM((_CHUNK,), jnp.int32),
            pltpu.VMEM((_CHUNK,), jnp.int32),
            pltpu.VMEM((_CHUNK, dw), jnp.float32),
            pltpu.VMEM((_CHUNK, dw), jnp.float32),
            pltpu.SemaphoreType.DMA,
            pltpu.SemaphoreType.DMA,
        ])
    def gk(ta, tb, ia, ib, out, ia_v, ib_v, a_v, b_v, s1, s2):
        wid = lax.axis_index("s") * 2 + lax.axis_index("c")
        base0 = wid * pw

        def chunk(j, carry):
            base = base0 + j * _CHUNK
            pltpu.sync_copy(ia.at[pl.ds(base, _CHUNK)], ia_v)
            pltpu.sync_copy(ib.at[pl.ds(base, _CHUNK)], ib_v)
            ca = pltpu.async_copy(ta.at[ia_v], a_v, s1)
            cb = pltpu.async_copy(tb.at[ib_v], b_v, s2)
            ca.wait()
            cb.wait()

            def row(r, c2):
                for c0 in range(0, dw, 16):
                    a_v[r, pl.ds(c0, 16)] = (
                        a_v[r, pl.ds(c0, 16)] + b_v[r, pl.ds(c0, 16)])
                return c2

            lax.fori_loop(0, _CHUNK, row, 0)
            pltpu.sync_copy(a_v, out.at[pl.ds(base, _CHUNK)])
            return carry

        lax.fori_loop(0, nch, chunk, 0)

    return gk(tab_a, tab_b, idx_a, idx_b)


def _sc_scatter_add(vals, idx, nn, ep):
    dv = vals.shape[1]
    assert dv == 128
    cw = 64
    pw = ep // _NW
    nch = pw // cw
    nn_pad = -(-nn // 128) * 128
    rs = nn_pad // 16

    @functools.partial(
        pl.kernel, mesh=_sc_mesh(),
        out_type=jax.ShapeDtypeStruct((2, nn_pad, dv), jnp.float32),
        scratch_types=[
            pltpu.VMEM((1, cw), jnp.int32),
            pltpu.VMEM((cw, dv), jnp.float32),
            pltpu.VMEM_SHARED((nn_pad, dv), jnp.float32),
        ])
    def sk(v_hbm, i_hbm, z_hbm, out, i_v, v_v, tab):
        c = lax.axis_index("c")
        s = lax.axis_index("s")
        pltpu.sync_copy(z_hbm.at[pl.ds(s * rs, rs)], tab.at[pl.ds(s * rs, rs)])
        plsc.subcore_barrier()
        base0 = (c * 16 + s) * pw

        def chunk(j, carry):
            base = base0 + j * cw
            pltpu.sync_copy(i_hbm.at[pl.ds(base, cw)], i_v.at[0])
            pltpu.sync_copy(v_hbm.at[pl.ds(base, cw)], v_v)
            pltpu.sync_copy(v_v, tab.at[i_v.at[0]], add=True)
            return carry

        lax.fori_loop(0, nch, chunk, 0)
        plsc.subcore_barrier()
        pltpu.sync_copy(tab.at[pl.ds(s * rs, rs)], out.at[c, pl.ds(s * rs, rs)])

    return sk(vals, idx, jnp.zeros((nn_pad, dv), jnp.float32))[:, :nn]


def _proj1_t(xt0, w_tt, b_tt, w_tp, b_tp, g08, br):
    consts = [w_tt[64:128], w_tt[128:192], w_tt[192:256], b_tt.reshape(1, -1),
              w_tp[64:128], w_tp[192:256], b_tp.reshape(1, -1), g08]

    def body(i, xs, cs):
        xb, = xs
        ws, wd, wg, b, ws2, wg2, b2, g = cs
        g0 = g[0:1, :]
        gt = 0.5 * (g0 @ wg + b)
        gt2 = 0.5 * (g0 @ wg2 + b2)
        return [xb @ ws + gt, xb @ wd + gt, xb @ ws2 + gt2]

    return _pcall(body, [xt0], consts, [('b', 128)] * 3, br)


def _proj1_p(xp0, w_tp, b_tp, g08, br):
    consts = [w_tp[128:192], w_tp[192:256], b_tp.reshape(1, -1), g08]

    def body(i, xs, cs):
        xb, = xs
        wd, wg, b, g = cs
        return [xb @ wd + 0.5 * (g[0:1, :] @ wg + b)]

    return _pcall(body, [xp0], consts, [('b', 128)], br)[0]


def _proj2_t(xt0, nt1, w_tt, b_tt, w_tp, b_tp, g08, g18, br):
    consts = [w_tt[128:192], w_tt[192:256], w_tt[256:320], w_tt[320:384],
              w_tt[384:448], w_tt[448:512], b_tt.reshape(1, -1),
              w_tp[128:192], w_tp[192:256],
              w_tp[384:448], w_tp[448:512], b_tp.reshape(1, -1),
              g08, g18]

    def body(i, xs, cs):
        xb, nb = xs
        (wsa, wsb, wda, wdb, wg0, wg1, b,
         ws2a, ws2b, wg20, wg21, b2, g0f, g1f) = cs
        g0 = g0f[0:1, :]
        g1 = g1f[0:1, :]
        gt = 0.5 * (g0 @ wg0 + g1 @ wg1 + b)
        gt2 = 0.5 * (g0 @ wg20 + g1 @ wg21 + b2)
        return [xb @ wsa + nb @ wsb + gt,
                xb @ wda + nb @ wdb + gt,
                xb @ ws2a + nb @ ws2b + gt2]

    return _pcall(body, [xt0, nt1], consts, [('b', 128)] * 3, br)


def _proj2_p(xp0, np1, w_tp, b_tp, g08, g18, br):
    consts = [w_tp[256:320], w_tp[320:384], w_tp[384:448], w_tp[448:512],
              b_tp.reshape(1, -1), g08, g18]

    def body(i, xs, cs):
        xb, nb = xs
        wda, wdb, wg0, wg1, b, g0f, g1f = cs
        gt = 0.5 * (g0f[0:1, :] @ wg0 + g1f[0:1, :] @ wg1 + b)
        return [xb @ wda + nb @ wdb + gt]

    return _pcall(body, [xp0, np1], consts, [('b', 128)], br)[0]


def _edge_block1(e0, gsum, ps, n_real, br):
    consts = [ps[0]['W'][0:64]] + _tail_consts(ps)

    def body(i, xs, cs):
        e0b, gb = xs
        it = iter(cs)
        z = e0b @ next(it) + gb
        h = _tail_chain(z, it, 4)
        rows = i * br + lax.broadcasted_iota(jnp.int32, (br, 1), 0)
        h = jnp.where(rows < n_real, h, 0.0)
        s = jnp.pad(jnp.sum(h, 0, keepdims=True), ((0, 7), (0, 0)))
        return [jnp.pad(h, ((0, 0), (0, 64))), s]

    return _pcall(body, [e0, gsum], consts, [('b', 128), ('s', 64)], br)


def _edge_block2(e0, e1, gsum, ps, dec_ps, w_out, b_out, n_real, br):
    w0 = ps[0]['W']
    consts = ([w0[0:64], w0[64:128]] + _tail_consts(ps)
              + _full_consts(dec_ps) + [w_out, b_out.reshape(1, -1)])
    dout = w_out.shape[1]

    def body(i, xs, cs):
        e0b, e1b, gb = xs
        it = iter(cs)
        z = e0b @ next(it) + e1b[:, :64] @ next(it) + gb
        h = _tail_chain(z, it, 4)
        rows = i * br + lax.broadcasted_iota(jnp.int32, (br, 1), 0)
        e2 = jnp.where(rows < n_real, h, 0.0)
        hd = _full_chain(h, it, 4)
        y = hd @ next(it) + next(it)
        return [jnp.pad(e2, ((0, 0), (0, 64))), y]

    return _pcall(body, [e0, e1, gsum], consts,
                  [('b', 128), ('b', dout)], br)


def _node_block1(x, agg2, ps, g08, br):
    w0 = ps[0]['W']
    consts = ([w0[0:64], w0[64:128], w0[128:192], ps[0]['b'].reshape(1, -1),
               g08] + _tail_consts(ps))

    def body(i, xs, cs):
        xb, a0, a1 = xs
        it = iter(cs)
        wx, wa, wg, b, g = next(it), next(it), next(it), next(it), next(it)
        z = xb @ wx + (a0 + a1)[:, :64] @ wa + (g[0:1, :] @ wg + b)
        h = _tail_chain(z, it, 4)
        s = jnp.pad(jnp.sum(h, 0, keepdims=True), ((0, 7), (0, 0)))
        return [h, s]

    return _pcall(body, [x, agg2[0], agg2[1]], consts,
                  [('b', 64), ('s', 64)], br)


def _node_block2(x0, n1, agg2, ps, dec_ps, g08, g18, br):
    w0 = ps[0]['W']
    consts = ([w0[0:64], w0[64:128], w0[128:192], w0[192:256], w0[256:320],
               ps[0]['b'].reshape(1, -1), g08, g18]
              + _tail_consts(ps) + _full_consts(dec_ps))

    def body(i, xs, cs):
        xb, nb, a0, a1 = xs
        it = iter(cs)
        wxa, wxb, wa, wg0, wg1, b, g0f, g1f = (
            next(it), next(it), next(it), next(it), next(it), next(it),
            next(it), next(it))
        z = (xb @ wxa + nb @ wxb + (a0 + a1)[:, :64] @ wa
             + (g0f[0:1, :] @ wg0 + g1f[0:1, :] @ wg1 + b))
        h = _tail_chain(z, it, 4)
        return [_full_chain(h, it, 4)]

    return _pcall(body, [x0, n1, agg2[0], agg2[1]], consts,
                  [('b', 64)], br)


def _global1(g08, s_ett, s_etp, s_nt, s_np, ps, inv_ett, inv_etp,
             inv_nt, inv_np):
    w0 = ps[0]['W']
    consts = ([s_ett, s_etp, s_nt, s_np,
               w0[0:64], w0[64:128], w0[128:192], w0[192:256], w0[256:320],
               ps[0]['b'].reshape(1, -1)] + _tail_consts(ps))

    def body(i, xs, cs):
        gb, = xs
        it = iter(cs)
        se, sp, sn, sq = next(it), next(it), next(it), next(it)
        wg, w1, w2, w3, w4, b = (next(it), next(it), next(it), next(it),
                                 next(it), next(it))
        z = (gb @ wg + (se[0:1, :] * inv_ett) @ w1
             + (sp[0:1, :] * inv_etp) @ w2 + (sn[0:1, :] * inv_nt) @ w3
             + (sq[0:1, :] * inv_np) @ w4 + b)
        return [_tail_chain(z, it, 4)]

    return _pcall(body, [g08], consts, [('b', 64)], 8)[0]


def _pad_rows(x, n):
    if x.shape[0] == n:
        return x
    return jnp.pad(x, ((0, n - x.shape[0]),) + ((0, 0),) * (x.ndim - 1))


def kernel(tracks_x, pvs_x, tt_edge_attr, tp_edge_attr, globals_x, params,
           tt_edge_index, tp_edge_index):
    P = params
    nt, npv = tracks_x.shape[0], pvs_x.shape[0]
    ett_n, etp_n = tt_edge_attr.shape[0], tp_edge_attr.shape[0]
    unit = _NW * _CHUNK
    ep_tt = -(-ett_n // unit) * unit
    ep_tp = -(-etp_n // unit) * unit

    tta = _pad_rows(tt_edge_attr, ep_tt)
    tpa = _pad_rows(tp_edge_attr, ep_tp)
    tts = _pad_rows(tt_edge_index[0], ep_tt)
    ttd = _pad_rows(tt_edge_index[1], ep_tt)
    tps = _pad_rows(tp_edge_index[0], ep_tp)
    tpd = _pad_rows(tp_edge_index[1], ep_tp)
    g_in = jnp.pad(globals_x, ((0, 7), (0, 0)))

    brn, bre = 400, 512

    xt0 = _run_mlp(tracks_x, P['enc_node_tracks'], brn)
    xp0 = _run_mlp(pvs_x, P['enc_node_pvs'], brn)
    ett0 = _run_mlp(tta, P['enc_edge_tt'], bre)
    etp0 = _run_mlp(tpa, P['enc_edge_tp'], bre)
    g08 = _run_mlp(g_in, P['enc_global'], 8)

    b1 = P['block1']
    w1tt, b1tt = b1['edge_tt'][0]['W'], b1['edge_tt'][0]['b']
    w1tp, b1tp = b1['edge_tp'][0]['W'], b1['edge_tp'][0]['b']
    ps_tt, pd_tt, ps_tp = _proj1_t(xt0, w1tt, b1tt, w1tp, b1tp, g08, brn)
    pd_tp = _proj1_p(xp0, w1tp, b1tp, g08, brn)

    g1_tt = _sc_gather_sum(ps_tt, pd_tt, tts, ttd, ep_tt)
    g1_tp = _sc_gather_sum(ps_tp, pd_tp, tps, tpd, ep_tp)

    e_tt1, s_ett1 = _edge_block1(ett0, g1_tt, b1['edge_tt'], ett_n, bre)
    e_tp1, s_etp1 = _edge_block1(etp0, g1_tp, b1['edge_tp'], etp_n, bre)

    agg_t1 = _sc_scatter_add(e_tt1, ttd, nt, ep_tt)
    agg_p1 = _sc_scatter_add(e_tp1, tpd, npv, ep_tp)

    n_t1, s_nt1 = _node_block1(xt0, agg_t1, b1['node_tracks'], g08, brn)
    n_p1, s_np1 = _node_block1(xp0, agg_p1, b1['node_pvs'], g08, brn)

    g18 = _global1(g08, s_ett1, s_etp1, s_nt1, s_np1, b1['global'],
                   1.0 / ett_n, 1.0 / etp_n, 1.0 / nt, 1.0 / npv)

    b2 = P['block2']
    w2tt, b2tt = b2['edge_tt'][0]['W'], b2['edge_tt'][0]['b']
    w2tp, b2tp = b2['edge_tp'][0]['W'], b2['edge_tp'][0]['b']
    qs_tt, qd_tt, qs_tp = _proj2_t(xt0, n_t1, w2tt, b2tt, w2tp, b2tp,
                                   g08, g18, brn)
    qd_tp = _proj2_p(xp0, n_p1, w2tp, b2tp, g08, g18, brn)

    g2_tt = _sc_gather_sum(qs_tt, qd_tt, tts, ttd, ep_tt)
    g2_tp = _sc_gather_sum(qs_tp, qd_tp, tps, tpd, ep_tp)

    e_tt2, o_tt = _edge_block2(ett0, e_tt1, g2_tt, b2['edge_tt'],
                               P['dec_edge_tt'], P['out_tt']['W'],
                               P['out_tt']['b'], ett_n, bre)
    e_tp2, o_tp = _edge_block2(etp0, e_tp1, g2_tp, b2['edge_tp'],
                               P['dec_edge_tp'], P['out_tp']['W'],
                               P['out_tp']['b'], etp_n, bre)

    agg_t2 = _sc_scatter_add(e_tt2, ttd, nt, ep_tt)
    agg_p2 = _sc_scatter_add(e_tp2, tpd, npv, ep_tp)

    dxt = _node_block2(xt0, n_t1, agg_t2, b2['node_tracks'],
                       P['dec_node_tracks'], g08, g18, brn)[0]
    dxp = _node_block2(xp0, n_p1, agg_p2, b2['node_pvs'],
                       P['dec_node_pvs'], g08, g18, brn)[0]

    return (o_tt[:ett_n], o_tp[:etp_n], dxt, dxp)


# speedup vs baseline: 1.6474x; 1.0701x over previous
import functools

import jax
import jax.numpy as jnp
from jax import lax
from jax.experimental import pallas as pl
from jax.experimental.pallas import tpu as pltpu
from jax.experimental.pallas import tpu_sc as plsc


def _pcall(body, blocked, consts, outs, br):
    n_rows = blocked[0].shape[0]
    grid = (n_rows // br,)
    in_specs = []
    for x in blocked:
        in_specs.append(pl.BlockSpec((br, x.shape[1]), lambda i: (i, 0)))
    for c in consts:
        in_specs.append(pl.BlockSpec(c.shape, lambda i: (0, 0)))
    out_specs, out_shapes = [], []
    for kind, d in outs:
        if kind == 'b':
            out_specs.append(pl.BlockSpec((br, d), lambda i: (i, 0)))
            out_shapes.append(jax.ShapeDtypeStruct((n_rows, d), jnp.float32))
        else:
            out_specs.append(pl.BlockSpec((8, d), lambda i: (0, 0)))
            out_shapes.append(jax.ShapeDtypeStruct((8, d), jnp.float32))
    nb, nc = len(blocked), len(consts)

    def kern(*refs):
        i = pl.program_id(0)
        xs = [refs[k][...] for k in range(nb)]
        cs = [refs[nb + k][...] for k in range(nc)]
        vals = body(i, xs, cs)
        orefs = refs[nb + nc:]
        for (kind, _d), ref, v in zip(outs, orefs, vals):
            if kind == 'b':
                ref[...] = v
            else:
                @pl.when(i == 0)
                def _init(ref=ref, v=v):
                    ref[...] = v

                @pl.when(i != 0)
                def _acc(ref=ref, v=v):
                    ref[...] = ref[...] + v

    return pl.pallas_call(
        kern, grid=grid, in_specs=in_specs, out_specs=out_specs,
        out_shape=out_shapes)(*blocked, *consts)


def _ln_relu(z, g, be):
    m = z.mean(-1, keepdims=True)
    v = jnp.mean((z - m) ** 2, -1, keepdims=True)
    return jax.nn.relu((z - m) * lax.rsqrt(v + 1e-5) * g + be)


def _tail_consts(ps):
    out = [ps[0]['g'].reshape(1, -1), ps[0]['be'].reshape(1, -1)]
    for p in ps[1:]:
        out += [p['W'], p['b'].reshape(1, -1)]
        if 'g' in p:
            out += [p['g'].reshape(1, -1), p['be'].reshape(1, -1)]
    return out


def _tail_chain(z, it, nl):
    h = _ln_relu(z, next(it), next(it))
    for k in range(1, nl):
        h = h @ next(it) + next(it)
        if k < nl - 1:
            h = _ln_relu(h, next(it), next(it))
    return h


def _full_consts(ps):
    out = []
    for p in ps:
        out += [p['W'], p['b'].reshape(1, -1)]
        if 'g' in p:
            out += [p['g'].reshape(1, -1), p['be'].reshape(1, -1)]
    return out


def _full_chain(h, it, nl):
    for k in range(nl):
        h = h @ next(it) + next(it)
        if k < nl - 1:
            h = _ln_relu(h, next(it), next(it))
    return h


def _run_mlp(x, ps, br):
    nl = len(ps)

    def body(i, xs, cs):
        return [_full_chain(xs[0], iter(cs), nl)]

    return _pcall(body, [x], _full_consts(ps),
                  [('b', ps[-1]['W'].shape[1])], br)[0]


_NW = 32
_CHUNK = 128


def _sc_mesh():
    return plsc.VectorSubcoreMesh(core_axis_name="c", subcore_axis_name="s")


def _sc_gather_sum(tab_a, tab_b, idx_a, idx_b, ep):
    dw = tab_a.shape[1]
    c = _CHUNK
    pw = ep // _NW
    nch = pw // c
    npair = nch // 2

    @functools.partial(
        pl.kernel, mesh=_sc_mesh(),
        out_type=jax.ShapeDtypeStruct((ep, dw), jnp.float32),
        scratch_types=[
            pltpu.VMEM((2, c), jnp.int32),
            pltpu.VMEM((2, c), jnp.int32),
            pltpu.VMEM((c, dw), jnp.float32),
            pltpu.VMEM((c, dw), jnp.float32),
            pltpu.VMEM((c, dw), jnp.float32),
            pltpu.VMEM((c, dw), jnp.float32),
            pltpu.VMEM((c, dw), jnp.float32),
            pltpu.VMEM((c, dw), jnp.float32),
        ] + [pltpu.SemaphoreType.DMA] * 6)
    def gk(ta, tb, ia, ib, out, iav, ibv, a0, a1, b0, b1, o0, o1,
           sa0, sa1, sb0, sb1, w0, w1):
        wid = lax.axis_index("s") * 2 + lax.axis_index("c")
        base0 = wid * pw
        av = (a0, a1)
        bv = (b0, b1)
        ov = (o0, o1)
        sa = (sa0, sa1)
        sb = (sb0, sb1)
        wv = (w0, w1)

        def fire(j, p):
            pltpu.sync_copy(ia.at[pl.ds(base0 + j * c, c)], iav.at[p])
            pltpu.sync_copy(ib.at[pl.ds(base0 + j * c, c)], ibv.at[p])
            pltpu.async_copy(ta.at[iav.at[p]], av[p], sa[p])
            pltpu.async_copy(tb.at[ibv.at[p]], bv[p], sb[p])

        def addstore(j, p):
            pltpu.make_async_copy(ta.at[iav.at[p]], av[p], sa[p]).wait()
            pltpu.make_async_copy(tb.at[ibv.at[p]], bv[p], sb[p]).wait()

            def row(r, c2):
                for c0 in range(0, dw, 16):
                    ov[p][r, pl.ds(c0, 16)] = (
                        av[p][r, pl.ds(c0, 16)] + bv[p][r, pl.ds(c0, 16)])
                return c2

            lax.fori_loop(0, c, row, 0)
            pltpu.async_copy(ov[p], out.at[pl.ds(base0 + j * c, c)], wv[p])

        def wait_wb(p):
            pltpu.make_async_copy(
                ov[p], out.at[pl.ds(base0, c)], wv[p]).wait()

        fire(0, 0)

        def pair(k, carry):
            j0 = 2 * k
            fire(j0 + 1, 1)

            @pl.when(k >= 1)
            def _():
                wait_wb(0)

            addstore(j0, 0)

            @pl.when(k < npair - 1)
            def _():
                fire(j0 + 2, 0)

            @pl.when(k >= 1)
            def _():
                wait_wb(1)

            addstore(j0 + 1, 1)
            return carry

        lax.fori_loop(0, npair, pair, 0)
        wait_wb(0)
        wait_wb(1)

    return gk(tab_a, tab_b, idx_a, idx_b)


def _sc_scatter_add(vals, idx, nn, ep):
    dv = vals.shape[1]
    assert dv == 128
    cw = 64
    pw = ep // _NW
    nch = pw // cw
    nn_pad = -(-nn // 128) * 128
    rs = nn_pad // 16

    @functools.partial(
        pl.kernel, mesh=_sc_mesh(),
        out_type=jax.ShapeDtypeStruct((2, nn_pad, dv), jnp.float32),
        scratch_types=[
            pltpu.VMEM((2, cw), jnp.int32),
            pltpu.VMEM((cw, dv), jnp.float32),
            pltpu.VMEM((cw, dv), jnp.float32),
            pltpu.VMEM_SHARED((nn_pad, dv), jnp.float32),
            pltpu.SemaphoreType.DMA,
            pltpu.SemaphoreType.DMA,
            pltpu.SemaphoreType.DMA,
            pltpu.SemaphoreType.DMA,
        ])
    def sk(v_hbm, i_hbm, z_hbm, out, i_v, v0, v1, tab, si0, si1, sv0, sv1):
        c = lax.axis_index("c")
        s = lax.axis_index("s")
        pltpu.sync_copy(z_hbm.at[pl.ds(s * rs, rs)], tab.at[pl.ds(s * rs, rs)])
        plsc.subcore_barrier()
        base0 = (c * 16 + s) * pw
        vv = (v0, v1)
        si = (si0, si1)
        sv = (sv0, sv1)

        def fire(j, p):
            base = base0 + j * cw
            pltpu.async_copy(i_hbm.at[pl.ds(base, cw)], i_v.at[p], si[p])
            pltpu.async_copy(v_hbm.at[pl.ds(base, cw)], vv[p], sv[p])

        def flush(j, p):
            base = base0 + j * cw
            pltpu.make_async_copy(
                i_hbm.at[pl.ds(base, cw)], i_v.at[p], si[p]).wait()
            pltpu.make_async_copy(
                v_hbm.at[pl.ds(base, cw)], vv[p], sv[p]).wait()
            pltpu.sync_copy(vv[p], tab.at[i_v.at[p]], add=True)

        fire(0, 0)

        def pair(k, carry):
            j0 = 2 * k
            fire(j0 + 1, 1)
            flush(j0, 0)

            @pl.when(k < nch // 2 - 1)
            def _():
                fire(j0 + 2, 0)

            flush(j0 + 1, 1)
            return carry

        lax.fori_loop(0, nch // 2, pair, 0)
        plsc.subcore_barrier()
        pltpu.sync_copy(tab.at[pl.ds(s * rs, rs)], out.at[c, pl.ds(s * rs, rs)])

    return sk(vals, idx, jnp.zeros((nn_pad, dv), jnp.float32))[:, :nn]


def _proj1_t(xt0, w_tt, b_tt, w_tp, b_tp, g08, br):
    consts = [w_tt[64:128], w_tt[128:192], w_tt[192:256], b_tt.reshape(1, -1),
              w_tp[64:128], w_tp[192:256], b_tp.reshape(1, -1), g08]

    def body(i, xs, cs):
        xb, = xs
        ws, wd, wg, b, ws2, wg2, b2, g = cs
        g0 = g[0:1, :]
        gt = 0.5 * (g0 @ wg + b)
        gt2 = 0.5 * (g0 @ wg2 + b2)
        return [xb @ ws + gt, xb @ wd + gt, xb @ ws2 + gt2]

    return _pcall(body, [xt0], consts, [('b', 128)] * 3, br)


def _proj1_p(xp0, w_tp, b_tp, g08, br):
    consts = [w_tp[128:192], w_tp[192:256], b_tp.reshape(1, -1), g08]

    def body(i, xs, cs):
        xb, = xs
        wd, wg, b, g = cs
        return [xb @ wd + 0.5 * (g[0:1, :] @ wg + b)]

    return _pcall(body, [xp0], consts, [('b', 128)], br)[0]


def _proj2_t(xt0, nt1, w_tt, b_tt, w_tp, b_tp, g08, g18, br):
    consts = [w_tt[128:192], w_tt[192:256], w_tt[256:320], w_tt[320:384],
              w_tt[384:448], w_tt[448:512], b_tt.reshape(1, -1),
              w_tp[128:192], w_tp[192:256],
              w_tp[384:448], w_tp[448:512], b_tp.reshape(1, -1),
              g08, g18]

    def body(i, xs, cs):
        xb, nb = xs
        (wsa, wsb, wda, wdb, wg0, wg1, b,
         ws2a, ws2b, wg20, wg21, b2, g0f, g1f) = cs
        g0 = g0f[0:1, :]
        g1 = g1f[0:1, :]
        gt = 0.5 * (g0 @ wg0 + g1 @ wg1 + b)
        gt2 = 0.5 * (g0 @ wg20 + g1 @ wg21 + b2)
        return [xb @ wsa + nb @ wsb + gt,
                xb @ wda + nb @ wdb + gt,
                xb @ ws2a + nb @ ws2b + gt2]

    return _pcall(body, [xt0, nt1], consts, [('b', 128)] * 3, br)


def _proj2_p(xp0, np1, w_tp, b_tp, g08, g18, br):
    consts = [w_tp[256:320], w_tp[320:384], w_tp[384:448], w_tp[448:512],
              b_tp.reshape(1, -1), g08, g18]

    def body(i, xs, cs):
        xb, nb = xs
        wda, wdb, wg0, wg1, b, g0f, g1f = cs
        gt = 0.5 * (g0f[0:1, :] @ wg0 + g1f[0:1, :] @ wg1 + b)
        return [xb @ wda + nb @ wdb + gt]

    return _pcall(body, [xp0, np1], consts, [('b', 128)], br)[0]


def _edge_block1(e0, gsum, ps, n_real, br):
    consts = [ps[0]['W'][0:64]] + _tail_consts(ps)

    def body(i, xs, cs):
        e0b, gb = xs
        it = iter(cs)
        z = e0b @ next(it) + gb
        h = _tail_chain(z, it, 4)
        rows = i * br + lax.broadcasted_iota(jnp.int32, (br, 1), 0)
        h = jnp.where(rows < n_real, h, 0.0)
        s = jnp.pad(jnp.sum(h, 0, keepdims=True), ((0, 7), (0, 0)))
        return [jnp.pad(h, ((0, 0), (0, 64))), s]

    return _pcall(body, [e0, gsum], consts, [('b', 128), ('s', 64)], br)


def _edge_block2(e0, e1, gsum, ps, dec_ps, w_out, b_out, n_real, br):
    w0 = ps[0]['W']
    consts = ([w0[0:64], w0[64:128]] + _tail_consts(ps)
              + _full_consts(dec_ps) + [w_out, b_out.reshape(1, -1)])
    dout = w_out.shape[1]

    def body(i, xs, cs):
        e0b, e1b, gb = xs
        it = iter(cs)
        z = e0b @ next(it) + e1b[:, :64] @ next(it) + gb
        h = _tail_chain(z, it, 4)
        rows = i * br + lax.broadcasted_iota(jnp.int32, (br, 1), 0)
        e2 = jnp.where(rows < n_real, h, 0.0)
        hd = _full_chain(h, it, 4)
        y = hd @ next(it) + next(it)
        return [jnp.pad(e2, ((0, 0), (0, 64))), y]

    return _pcall(body, [e0, e1, gsum], consts,
                  [('b', 128), ('b', dout)], br)


def _node_block1(x, agg2, ps, g08, br):
    w0 = ps[0]['W']
    consts = ([w0[0:64], w0[64:128], w0[128:192], ps[0]['b'].reshape(1, -1),
               g08] + _tail_consts(ps))

    def body(i, xs, cs):
        xb, a0, a1 = xs
        it = iter(cs)
        wx, wa, wg, b, g = next(it), next(it), next(it), next(it), next(it)
        z = xb @ wx + (a0 + a1)[:, :64] @ wa + (g[0:1, :] @ wg + b)
        h = _tail_chain(z, it, 4)
        s = jnp.pad(jnp.sum(h, 0, keepdims=True), ((0, 7), (0, 0)))
        return [h, s]

    return _pcall(body, [x, agg2[0], agg2[1]], consts,
                  [('b', 64), ('s', 64)], br)


def _node_block2(x0, n1, agg2, ps, dec_ps, g08, g18, br):
    w0 = ps[0]['W']
    consts = ([w0[0:64], w0[64:128], w0[128:192], w0[192:256], w0[256:320],
               ps[0]['b'].reshape(1, -1), g08, g18]
              + _tail_consts(ps) + _full_consts(dec_ps))

    def body(i, xs, cs):
        xb, nb, a0, a1 = xs
        it = iter(cs)
        wxa, wxb, wa, wg0, wg1, b, g0f, g1f = (
            next(it), next(it), next(it), next(it), next(it), next(it),
            next(it), next(it))
        z = (xb @ wxa + nb @ wxb + (a0 + a1)[:, :64] @ wa
             + (g0f[0:1, :] @ wg0 + g1f[0:1, :] @ wg1 + b))
        h = _tail_chain(z, it, 4)
        return [_full_chain(h, it, 4)]

    return _pcall(body, [x0, n1, agg2[0], agg2[1]], consts,
                  [('b', 64)], br)


def _global1(g08, s_ett, s_etp, s_nt, s_np, ps, inv_ett, inv_etp,
             inv_nt, inv_np):
    w0 = ps[0]['W']
    consts = ([s_ett, s_etp, s_nt, s_np,
               w0[0:64], w0[64:128], w0[128:192], w0[192:256], w0[256:320],
               ps[0]['b'].reshape(1, -1)] + _tail_consts(ps))

    def body(i, xs, cs):
        gb, = xs
        it = iter(cs)
        se, sp, sn, sq = next(it), next(it), next(it), next(it)
        wg, w1, w2, w3, w4, b = (next(it), next(it), next(it), next(it),
                                 next(it), next(it))
        z = (gb @ wg + (se[0:1, :] * inv_ett) @ w1
             + (sp[0:1, :] * inv_etp) @ w2 + (sn[0:1, :] * inv_nt) @ w3
             + (sq[0:1, :] * inv_np) @ w4 + b)
        return [_tail_chain(z, it, 4)]

    return _pcall(body, [g08], consts, [('b', 64)], 8)[0]


def _pad_rows(x, n):
    if x.shape[0] == n:
        return x
    return jnp.pad(x, ((0, n - x.shape[0]),) + ((0, 0),) * (x.ndim - 1))


def kernel(tracks_x, pvs_x, tt_edge_attr, tp_edge_attr, globals_x, params,
           tt_edge_index, tp_edge_index):
    P = params
    nt, npv = tracks_x.shape[0], pvs_x.shape[0]
    ett_n, etp_n = tt_edge_attr.shape[0], tp_edge_attr.shape[0]
    unit = _NW * _CHUNK
    ep_tt = -(-ett_n // unit) * unit
    ep_tp = -(-etp_n // unit) * unit

    tta = _pad_rows(tt_edge_attr, ep_tt)
    tpa = _pad_rows(tp_edge_attr, ep_tp)
    tts = _pad_rows(tt_edge_index[0], ep_tt)
    ttd = _pad_rows(tt_edge_index[1], ep_tt)
    tps = _pad_rows(tp_edge_index[0], ep_tp)
    tpd = _pad_rows(tp_edge_index[1], ep_tp)
    g_in = jnp.pad(globals_x, ((0, 7), (0, 0)))

    brn, bre = 400, 512

    xt0 = _run_mlp(tracks_x, P['enc_node_tracks'], brn)
    xp0 = _run_mlp(pvs_x, P['enc_node_pvs'], brn)
    ett0 = _run_mlp(tta, P['enc_edge_tt'], bre)
    etp0 = _run_mlp(tpa, P['enc_edge_tp'], bre)
    g08 = _run_mlp(g_in, P['enc_global'], 8)

    b1 = P['block1']
    w1tt, b1tt = b1['edge_tt'][0]['W'], b1['edge_tt'][0]['b']
    w1tp, b1tp = b1['edge_tp'][0]['W'], b1['edge_tp'][0]['b']
    ps_tt, pd_tt, ps_tp = _proj1_t(xt0, w1tt, b1tt, w1tp, b1tp, g08, brn)
    pd_tp = _proj1_p(xp0, w1tp, b1tp, g08, brn)

    g1_tt = _sc_gather_sum(ps_tt, pd_tt, tts, ttd, ep_tt)
    g1_tp = _sc_gather_sum(ps_tp, pd_tp, tps, tpd, ep_tp)

    e_tt1, s_ett1 = _edge_block1(ett0, g1_tt, b1['edge_tt'], ett_n, bre)
    e_tp1, s_etp1 = _edge_block1(etp0, g1_tp, b1['edge_tp'], etp_n, bre)

    agg_t1 = _sc_scatter_add(e_tt1, ttd, nt, ep_tt)
    agg_p1 = _sc_scatter_add(e_tp1, tpd, npv, ep_tp)

    n_t1, s_nt1 = _node_block1(xt0, agg_t1, b1['node_tracks'], g08, brn)
    n_p1, s_np1 = _node_block1(xp0, agg_p1, b1['node_pvs'], g08, brn)

    g18 = _global1(g08, s_ett1, s_etp1, s_nt1, s_np1, b1['global'],
                   1.0 / ett_n, 1.0 / etp_n, 1.0 / nt, 1.0 / npv)

    b2 = P['block2']
    w2tt, b2tt = b2['edge_tt'][0]['W'], b2['edge_tt'][0]['b']
    w2tp, b2tp = b2['edge_tp'][0]['W'], b2['edge_tp'][0]['b']
    qs_tt, qd_tt, qs_tp = _proj2_t(xt0, n_t1, w2tt, b2tt, w2tp, b2tp,
                                   g08, g18, brn)
    qd_tp = _proj2_p(xp0, n_p1, w2tp, b2tp, g08, g18, brn)

    g2_tt = _sc_gather_sum(qs_tt, qd_tt, tts, ttd, ep_tt)
    g2_tp = _sc_gather_sum(qs_tp, qd_tp, tps, tpd, ep_tp)

    e_tt2, o_tt = _edge_block2(ett0, e_tt1, g2_tt, b2['edge_tt'],
                               P['dec_edge_tt'], P['out_tt']['W'],
                               P['out_tt']['b'], ett_n, bre)
    e_tp2, o_tp = _edge_block2(etp0, e_tp1, g2_tp, b2['edge_tp'],
                               P['dec_edge_tp'], P['out_tp']['W'],
                               P['out_tp']['b'], etp_n, bre)

    agg_t2 = _sc_scatter_add(e_tt2, ttd, nt, ep_tt)
    agg_p2 = _sc_scatter_add(e_tp2, tpd, npv, ep_tp)

    dxt = _node_block2(xt0, n_t1, agg_t2, b2['node_tracks'],
                       P['dec_node_tracks'], g08, g18, brn)[0]
    dxp = _node_block2(xp0, n_p1, agg_p2, b2['node_pvs'],
                       P['dec_node_pvs'], g08, g18, brn)[0]

    return (o_tt[:ett_n], o_tp[:etp_n], dxt, dxp)


# speedup vs baseline: 2.1789x; 1.3226x over previous
import functools

import jax
import jax.numpy as jnp
from jax import lax
from jax.experimental import pallas as pl
from jax.experimental.pallas import tpu as pltpu
from jax.experimental.pallas import tpu_sc as plsc


def _pcall(body, blocked, consts, outs, br):
    n_rows = blocked[0].shape[0]
    grid = (n_rows // br,)
    in_specs = []
    for x in blocked:
        in_specs.append(pl.BlockSpec((br, x.shape[1]), lambda i: (i, 0)))
    for c in consts:
        in_specs.append(pl.BlockSpec(c.shape, lambda i: (0, 0)))
    out_specs, out_shapes = [], []
    for kind, d in outs:
        if kind == 'b':
            out_specs.append(pl.BlockSpec((br, d), lambda i: (i, 0)))
            out_shapes.append(jax.ShapeDtypeStruct((n_rows, d), jnp.float32))
        else:
            out_specs.append(pl.BlockSpec((8, d), lambda i: (0, 0)))
            out_shapes.append(jax.ShapeDtypeStruct((8, d), jnp.float32))
    nb, nc = len(blocked), len(consts)

    def kern(*refs):
        i = pl.program_id(0)
        xs = [refs[k][...] for k in range(nb)]
        cs = [refs[nb + k][...] for k in range(nc)]
        vals = body(i, xs, cs)
        orefs = refs[nb + nc:]
        for (kind, _d), ref, v in zip(outs, orefs, vals):
            if kind == 'b':
                ref[...] = v
            else:
                @pl.when(i == 0)
                def _init(ref=ref, v=v):
                    ref[...] = v

                @pl.when(i != 0)
                def _acc(ref=ref, v=v):
                    ref[...] = ref[...] + v

    return pl.pallas_call(
        kern, grid=grid, in_specs=in_specs, out_specs=out_specs,
        out_shape=out_shapes)(*blocked, *consts)


def _ln_relu(z, g, be):
    m = z.mean(-1, keepdims=True)
    v = jnp.mean((z - m) ** 2, -1, keepdims=True)
    return jax.nn.relu((z - m) * lax.rsqrt(v + 1e-5) * g + be)


def _tail_consts(ps):
    out = [ps[0]['g'].reshape(1, -1), ps[0]['be'].reshape(1, -1)]
    for p in ps[1:]:
        out += [p['W'], p['b'].reshape(1, -1)]
        if 'g' in p:
            out += [p['g'].reshape(1, -1), p['be'].reshape(1, -1)]
    return out


def _tail_chain(z, it, nl):
    h = _ln_relu(z, next(it), next(it))
    for k in range(1, nl):
        h = h @ next(it) + next(it)
        if k < nl - 1:
            h = _ln_relu(h, next(it), next(it))
    return h


def _full_consts(ps):
    out = []
    for p in ps:
        out += [p['W'], p['b'].reshape(1, -1)]
        if 'g' in p:
            out += [p['g'].reshape(1, -1), p['be'].reshape(1, -1)]
    return out


def _full_chain(h, it, nl):
    for k in range(nl):
        h = h @ next(it) + next(it)
        if k < nl - 1:
            h = _ln_relu(h, next(it), next(it))
    return h


def _run_mlp(x, ps, br):
    nl = len(ps)

    def body(i, xs, cs):
        return [_full_chain(xs[0], iter(cs), nl)]

    return _pcall(body, [x], _full_consts(ps),
                  [('b', ps[-1]['W'].shape[1])], br)[0]


_NW = 32
_CHUNK = 128


def _sc_mesh():
    return plsc.VectorSubcoreMesh(core_axis_name="c", subcore_axis_name="s")


def _sc_gather_sum2(ta1, tb1, ia1, ib1, ep1, ta2, tb2, ia2, ib2, ep2):
    dw = ta1.shape[1]
    c = _CHUNK

    @functools.partial(
        pl.kernel, mesh=_sc_mesh(),
        out_type=[jax.ShapeDtypeStruct((ep1, dw), jnp.float32),
                  jax.ShapeDtypeStruct((ep2, dw), jnp.float32)],
        scratch_types=[
            pltpu.VMEM((2, c), jnp.int32),
            pltpu.VMEM((2, c), jnp.int32),
            pltpu.VMEM((c, dw), jnp.float32),
            pltpu.VMEM((c, dw), jnp.float32),
            pltpu.VMEM((c, dw), jnp.float32),
            pltpu.VMEM((c, dw), jnp.float32),
            pltpu.VMEM((c, dw), jnp.float32),
            pltpu.VMEM((c, dw), jnp.float32),
        ] + [pltpu.SemaphoreType.DMA] * 6)
    def gk(ta1r, tb1r, ia1r, ib1r, ta2r, tb2r, ia2r, ib2r, out1, out2,
           iav, ibv, a0, a1, b0, b1, o0, o1, sa0, sa1, sb0, sb1, w0, w1):
        wid = lax.axis_index("s") * 2 + lax.axis_index("c")
        av = (a0, a1)
        bv = (b0, b1)
        ov = (o0, o1)
        sa = (sa0, sa1)
        sb = (sb0, sb1)
        wv = (w0, w1)

        def phase(ta, tb, ia, ib, out, ep):
            pw = ep // _NW
            nch = pw // c
            npair = nch // 2
            base0 = wid * pw

            def fire(j, p):
                pltpu.sync_copy(ia.at[pl.ds(base0 + j * c, c)], iav.at[p])
                pltpu.sync_copy(ib.at[pl.ds(base0 + j * c, c)], ibv.at[p])
                pltpu.async_copy(ta.at[iav.at[p]], av[p], sa[p])
                pltpu.async_copy(tb.at[ibv.at[p]], bv[p], sb[p])

            def addstore(j, p):
                pltpu.make_async_copy(ta.at[iav.at[p]], av[p], sa[p]).wait()
                pltpu.make_async_copy(tb.at[ibv.at[p]], bv[p], sb[p]).wait()

                def row(r, c2):
                    for c0 in range(0, dw, 16):
                        ov[p][r, pl.ds(c0, 16)] = (
                            av[p][r, pl.ds(c0, 16)] + bv[p][r, pl.ds(c0, 16)])
                    return c2

                lax.fori_loop(0, c, row, 0)
                pltpu.async_copy(ov[p], out.at[pl.ds(base0 + j * c, c)],
                                 wv[p])

            def wait_wb(p):
                pltpu.make_async_copy(
                    ov[p], out.at[pl.ds(base0, c)], wv[p]).wait()

            fire(0, 0)

            def pair(k, carry):
                j0 = 2 * k
                fire(j0 + 1, 1)

                @pl.when(k >= 1)
                def _():
                    wait_wb(0)

                addstore(j0, 0)

                @pl.when(k < npair - 1)
                def _():
                    fire(j0 + 2, 0)

                @pl.when(k >= 1)
                def _():
                    wait_wb(1)

                addstore(j0 + 1, 1)
                return carry

            lax.fori_loop(0, npair, pair, 0)
            wait_wb(0)
            wait_wb(1)

        phase(ta1r, tb1r, ia1r, ib1r, out1, ep1)
        phase(ta2r, tb2r, ia2r, ib2r, out2, ep2)

    return gk(ta1, tb1, ia1, ib1, ta2, tb2, ia2, ib2)


def _sc_scatter_add2(vals1, idx1, nn1, ep1, vals2, idx2, nn2, ep2):
    dv = vals1.shape[1]
    assert dv == 128 and vals2.shape[1] == 128
    cw = 64
    np1 = -(-nn1 // 128) * 128
    np2 = -(-nn2 // 128) * 128
    rs1 = np1 // 16
    rs2 = np2 // 16

    @functools.partial(
        pl.kernel, mesh=_sc_mesh(),
        out_type=[jax.ShapeDtypeStruct((2, np1, dv), jnp.float32),
                  jax.ShapeDtypeStruct((2, np2, dv), jnp.float32)],
        scratch_types=[
            pltpu.VMEM((2, cw), jnp.int32),
            pltpu.VMEM((cw, dv), jnp.float32),
            pltpu.VMEM((cw, dv), jnp.float32),
            pltpu.VMEM_SHARED((np1, dv), jnp.float32),
            pltpu.VMEM_SHARED((np2, dv), jnp.float32),
            pltpu.SemaphoreType.DMA,
            pltpu.SemaphoreType.DMA,
            pltpu.SemaphoreType.DMA,
            pltpu.SemaphoreType.DMA,
        ])
    def sk(v1, i1, v2, i2, z1, z2, out1, out2, i_v, b0, b1, tab1, tab2,
           si0, si1, sv0, sv1):
        cc = lax.axis_index("c")
        s = lax.axis_index("s")
        pltpu.sync_copy(z1.at[pl.ds(s * rs1, rs1)],
                        tab1.at[pl.ds(s * rs1, rs1)])
        pltpu.sync_copy(z2.at[pl.ds(s * rs2, rs2)],
                        tab2.at[pl.ds(s * rs2, rs2)])
        plsc.subcore_barrier()
        vv = (b0, b1)
        si = (si0, si1)
        sv = (sv0, sv1)

        def phase(v_hbm, i_hbm, tab, ep):
            pw = ep // _NW
            nch = pw // cw
            base0 = (cc * 16 + s) * pw

            def fire(j, p):
                base = base0 + j * cw
                pltpu.async_copy(i_hbm.at[pl.ds(base, cw)], i_v.at[p], si[p])
                pltpu.async_copy(v_hbm.at[pl.ds(base, cw)], vv[p], sv[p])

            def flush(j, p):
                base = base0 + j * cw
                pltpu.make_async_copy(
                    i_hbm.at[pl.ds(base, cw)], i_v.at[p], si[p]).wait()
                pltpu.make_async_copy(
                    v_hbm.at[pl.ds(base, cw)], vv[p], sv[p]).wait()
                pltpu.sync_copy(vv[p], tab.at[i_v.at[p]], add=True)

            fire(0, 0)

            def pair(k, carry):
                j0 = 2 * k
                fire(j0 + 1, 1)
                flush(j0, 0)

                @pl.when(k < nch // 2 - 1)
                def _():
                    fire(j0 + 2, 0)

                flush(j0 + 1, 1)
                return carry

            lax.fori_loop(0, nch // 2, pair, 0)

        phase(v1, i1, tab1, ep1)
        phase(v2, i2, tab2, ep2)
        plsc.subcore_barrier()
        pltpu.sync_copy(tab1.at[pl.ds(s * rs1, rs1)],
                        out1.at[cc, pl.ds(s * rs1, rs1)])
        pltpu.sync_copy(tab2.at[pl.ds(s * rs2, rs2)],
                        out2.at[cc, pl.ds(s * rs2, rs2)])

    o1, o2 = sk(vals1, idx1, vals2, idx2,
                jnp.zeros((np1, dv), jnp.float32),
                jnp.zeros((np2, dv), jnp.float32))
    return o1[:, :nn1], o2[:, :nn2]


def _enc_proj(x, ps, head_ws, g08, br):
    nl = len(ps)
    consts = _full_consts(ps) + [g08]
    for (wx, wg, b) in head_ws:
        consts += [wx, wg, b.reshape(1, -1)]

    def body(i, xs, cs):
        it = iter(cs)
        h = _full_chain(xs[0], it, nl)
        g = next(it)
        outs = [h]
        for _ in head_ws:
            wx, wg, b = next(it), next(it), next(it)
            outs.append(h @ wx + 0.5 * (g[0:1, :] @ wg + b))
        return outs

    return _pcall(body, [x], consts,
                  [('b', ps[-1]['W'].shape[1])] + [('b', 128)] * len(head_ws),
                  br)


def _edge_block1(e0, gsum, ps, n_real, br):
    consts = [ps[0]['W'][0:64]] + _tail_consts(ps)

    def body(i, xs, cs):
        e0b, gb = xs
        it = iter(cs)
        z = e0b @ next(it) + gb
        h = _tail_chain(z, it, 4)
        rows = i * br + lax.broadcasted_iota(jnp.int32, (br, 1), 0)
        h = jnp.where(rows < n_real, h, 0.0)
        s = jnp.pad(jnp.sum(h, 0, keepdims=True), ((0, 7), (0, 0)))
        return [jnp.pad(h, ((0, 0), (0, 64))), s]

    return _pcall(body, [e0, gsum], consts, [('b', 128), ('s', 64)], br)


def _edge_block2(e0, e1, gsum, gt8, ps, dec_ps, w_out, b_out, n_real, br):
    w0 = ps[0]['W']
    consts = ([w0[0:64], w0[64:128], gt8] + _tail_consts(ps)
              + _full_consts(dec_ps) + [w_out, b_out.reshape(1, -1)])
    dout = w_out.shape[1]

    def body(i, xs, cs):
        e0b, e1b, gb = xs
        it = iter(cs)
        z = e0b @ next(it) + e1b[:, :64] @ next(it) + gb + next(it)[0:1, :]
        h = _tail_chain(z, it, 4)
        rows = i * br + lax.broadcasted_iota(jnp.int32, (br, 1), 0)
        e2 = jnp.where(rows < n_real, h, 0.0)
        hd = _full_chain(h, it, 4)
        y = hd @ next(it) + next(it)
        return [jnp.pad(e2, ((0, 0), (0, 64))), y]

    return _pcall(body, [e0, e1, gsum], consts,
                  [('b', 128), ('b', dout)], br)


def _node_block1(x, agg2, ps, g08, head_ws, br):
    w0 = ps[0]['W']
    consts = ([w0[0:64], w0[64:128], w0[128:192], ps[0]['b'].reshape(1, -1),
               g08] + _tail_consts(ps))
    for (wx, wn) in head_ws:
        consts += [wx, wn]

    def body(i, xs, cs):
        xb, a0, a1 = xs
        it = iter(cs)
        wx, wa, wg, b, g = next(it), next(it), next(it), next(it), next(it)
        z = xb @ wx + (a0 + a1)[:, :64] @ wa + (g[0:1, :] @ wg + b)
        h = _tail_chain(z, it, 4)
        s = jnp.pad(jnp.sum(h, 0, keepdims=True), ((0, 7), (0, 0)))
        outs = [h, s]
        for _ in head_ws:
            whx, whn = next(it), next(it)
            outs.append(xb @ whx + h @ whn)
        return outs

    return _pcall(body, [x, agg2[0], agg2[1]], consts,
                  [('b', 64), ('s', 64)] + [('b', 128)] * len(head_ws), br)


def _node_block2(x0, n1, agg2, ps, dec_ps, g08, g18, br):
    w0 = ps[0]['W']
    consts = ([w0[0:64], w0[64:128], w0[128:192], w0[192:256], w0[256:320],
               ps[0]['b'].reshape(1, -1), g08, g18]
              + _tail_consts(ps) + _full_consts(dec_ps))

    def body(i, xs, cs):
        xb, nb, a0, a1 = xs
        it = iter(cs)
        wxa, wxb, wa, wg0, wg1, b, g0f, g1f = (
            next(it), next(it), next(it), next(it), next(it), next(it),
            next(it), next(it))
        z = (xb @ wxa + nb @ wxb + (a0 + a1)[:, :64] @ wa
             + (g0f[0:1, :] @ wg0 + g1f[0:1, :] @ wg1 + b))
        h = _tail_chain(z, it, 4)
        return [_full_chain(h, it, 4)]

    return _pcall(body, [x0, n1, agg2[0], agg2[1]], consts,
                  [('b', 64)], br)


def _global1(g08, s_ett, s_etp, s_nt, s_np, ps, inv_ett, inv_etp,
             inv_nt, inv_np, gterm_ws):
    w0 = ps[0]['W']
    consts = ([s_ett, s_etp, s_nt, s_np,
               w0[0:64], w0[64:128], w0[128:192], w0[192:256], w0[256:320],
               ps[0]['b'].reshape(1, -1)] + _tail_consts(ps))
    for (wg0, wg1, b) in gterm_ws:
        consts += [wg0, wg1, b.reshape(1, -1)]

    def body(i, xs, cs):
        gb, = xs
        it = iter(cs)
        se, sp, sn, sq = next(it), next(it), next(it), next(it)
        wg, w1, w2, w3, w4, b = (next(it), next(it), next(it), next(it),
                                 next(it), next(it))
        z = (gb @ wg + (se[0:1, :] * inv_ett) @ w1
             + (sp[0:1, :] * inv_etp) @ w2 + (sn[0:1, :] * inv_nt) @ w3
             + (sq[0:1, :] * inv_np) @ w4 + b)
        g1 = _tail_chain(z, it, 4)
        g1r = g1[0:1, :]
        g0r = gb[0:1, :]
        outs = [g1]
        for _ in gterm_ws:
            wg0, wg1_, bb = next(it), next(it), next(it)
            outs.append(jnp.broadcast_to(g0r @ wg0 + g1r @ wg1_ + bb,
                                         (8, 128)))
        return outs

    return _pcall(body, [g08], consts,
                  [('b', 64)] + [('b', 128)] * len(gterm_ws), 8)


def _pad_rows(x, n):
    if x.shape[0] == n:
        return x
    return jnp.pad(x, ((0, n - x.shape[0]),) + ((0, 0),) * (x.ndim - 1))


def kernel(tracks_x, pvs_x, tt_edge_attr, tp_edge_attr, globals_x, params,
           tt_edge_index, tp_edge_index):
    P = params
    nt, npv = tracks_x.shape[0], pvs_x.shape[0]
    ett_n, etp_n = tt_edge_attr.shape[0], tp_edge_attr.shape[0]
    unit = _NW * _CHUNK
    ep_tt = -(-ett_n // unit) * unit
    ep_tp = -(-etp_n // unit) * unit

    tta = _pad_rows(tt_edge_attr, ep_tt)
    tpa = _pad_rows(tp_edge_attr, ep_tp)
    tts = _pad_rows(tt_edge_index[0], ep_tt)
    ttd = _pad_rows(tt_edge_index[1], ep_tt)
    tps = _pad_rows(tp_edge_index[0], ep_tp)
    tpd = _pad_rows(tp_edge_index[1], ep_tp)
    g_in = jnp.pad(globals_x, ((0, 7), (0, 0)))

    brn, bre = 1000, 1024

    b1 = P['block1']
    b2 = P['block2']
    w1tt, b1tt = b1['edge_tt'][0]['W'], b1['edge_tt'][0]['b']
    w1tp, b1tp = b1['edge_tp'][0]['W'], b1['edge_tp'][0]['b']
    w2tt, b2tt = b2['edge_tt'][0]['W'], b2['edge_tt'][0]['b']
    w2tp, b2tp = b2['edge_tp'][0]['W'], b2['edge_tp'][0]['b']

    g08 = _run_mlp(g_in, P['enc_global'], 8)
    xt0, ps_tt, pd_tt, ps_tp = _enc_proj(
        tracks_x, P['enc_node_tracks'],
        [(w1tt[64:128], w1tt[192:256], b1tt),
         (w1tt[128:192], w1tt[192:256], b1tt),
         (w1tp[64:128], w1tp[192:256], b1tp)], g08, brn)
    xp0, pd_tp = _enc_proj(
        pvs_x, P['enc_node_pvs'],
        [(w1tp[128:192], w1tp[192:256], b1tp)], g08, brn)
    ett0 = _run_mlp(tta, P['enc_edge_tt'], bre)
    etp0 = _run_mlp(tpa, P['enc_edge_tp'], bre)

    g1_tt, g1_tp = _sc_gather_sum2(ps_tt, pd_tt, tts, ttd, ep_tt,
                                   ps_tp, pd_tp, tps, tpd, ep_tp)

    e_tt1, s_ett1 = _edge_block1(ett0, g1_tt, b1['edge_tt'], ett_n, bre)
    e_tp1, s_etp1 = _edge_block1(etp0, g1_tp, b1['edge_tp'], etp_n, bre)

    agg_t1, agg_p1 = _sc_scatter_add2(e_tt1, ttd, nt, ep_tt,
                                      e_tp1, tpd, npv, ep_tp)

    n_t1, s_nt1, qs_tt, qd_tt, qs_tp = _node_block1(
        xt0, agg_t1, b1['node_tracks'], g08,
        [(w2tt[128:192], w2tt[192:256]),
         (w2tt[256:320], w2tt[320:384]),
         (w2tp[128:192], w2tp[192:256])], brn)
    n_p1, s_np1, qd_tp = _node_block1(
        xp0, agg_p1, b1['node_pvs'], g08,
        [(w2tp[256:320], w2tp[320:384])], brn)

    g18, gt_tt, gt_tp = _global1(
        g08, s_ett1, s_etp1, s_nt1, s_np1, b1['global'],
        1.0 / ett_n, 1.0 / etp_n, 1.0 / nt, 1.0 / npv,
        [(w2tt[384:448], w2tt[448:512], b2tt),
         (w2tp[384:448], w2tp[448:512], b2tp)])

    g2_tt, g2_tp = _sc_gather_sum2(qs_tt, qd_tt, tts, ttd, ep_tt,
                                   qs_tp, qd_tp, tps, tpd, ep_tp)

    e_tt2, o_tt = _edge_block2(ett0, e_tt1, g2_tt, gt_tt, b2['edge_tt'],
                               P['dec_edge_tt'], P['out_tt']['W'],
                               P['out_tt']['b'], ett_n, bre)
    e_tp2, o_tp = _edge_block2(etp0, e_tp1, g2_tp, gt_tp, b2['edge_tp'],
                               P['dec_edge_tp'], P['out_tp']['W'],
                               P['out_tp']['b'], etp_n, bre)

    agg_t2, agg_p2 = _sc_scatter_add2(e_tt2, ttd, nt, ep_tt,
                                      e_tp2, tpd, npv, ep_tp)

    dxt = _node_block2(xt0, n_t1, agg_t2, b2['node_tracks'],
                       P['dec_node_tracks'], g08, g18, brn)[0]
    dxp = _node_block2(xp0, n_p1, agg_p2, b2['node_pvs'],
                       P['dec_node_pvs'], g08, g18, brn)[0]

    return (o_tt[:ett_n], o_tp[:etp_n], dxt, dxp)


# speedup vs baseline: 2.1944x; 1.0071x over previous
import functools

import jax
import jax.numpy as jnp
from jax import lax
from jax.experimental import pallas as pl
from jax.experimental.pallas import tpu as pltpu
from jax.experimental.pallas import tpu_sc as plsc


def _pcall(body, blocked, consts, outs, br):
    n_rows = blocked[0].shape[0]
    grid = (n_rows // br,)
    in_specs = []
    for x in blocked:
        in_specs.append(pl.BlockSpec((br, x.shape[1]), lambda i: (i, 0)))
    for c in consts:
        in_specs.append(pl.BlockSpec(c.shape, lambda i: (0, 0)))
    out_specs, out_shapes = [], []
    for kind, d in outs:
        if kind == 'b':
            out_specs.append(pl.BlockSpec((br, d), lambda i: (i, 0)))
            out_shapes.append(jax.ShapeDtypeStruct((n_rows, d), jnp.float32))
        else:
            out_specs.append(pl.BlockSpec((8, d), lambda i: (0, 0)))
            out_shapes.append(jax.ShapeDtypeStruct((8, d), jnp.float32))
    nb, nc = len(blocked), len(consts)

    def kern(*refs):
        i = pl.program_id(0)
        xs = [refs[k][...] for k in range(nb)]
        cs = [refs[nb + k][...] for k in range(nc)]
        vals = body(i, xs, cs)
        orefs = refs[nb + nc:]
        for (kind, _d), ref, v in zip(outs, orefs, vals):
            if kind == 'b':
                ref[...] = v
            else:
                @pl.when(i == 0)
                def _init(ref=ref, v=v):
                    ref[...] = v

                @pl.when(i != 0)
                def _acc(ref=ref, v=v):
                    ref[...] = ref[...] + v

    return pl.pallas_call(
        kern, grid=grid, in_specs=in_specs, out_specs=out_specs,
        out_shape=out_shapes)(*blocked, *consts)


def _ln_relu(z, g, be):
    m = z.mean(-1, keepdims=True)
    v = jnp.mean((z - m) ** 2, -1, keepdims=True)
    return jax.nn.relu((z - m) * lax.rsqrt(v + 1e-5) * g + be)


def _tail_consts(ps):
    out = [ps[0]['g'].reshape(1, -1), ps[0]['be'].reshape(1, -1)]
    for p in ps[1:]:
        out += [p['W'], p['b'].reshape(1, -1)]
        if 'g' in p:
            out += [p['g'].reshape(1, -1), p['be'].reshape(1, -1)]
    return out


def _tail_chain(z, it, nl):
    h = _ln_relu(z, next(it), next(it))
    for k in range(1, nl):
        h = h @ next(it) + next(it)
        if k < nl - 1:
            h = _ln_relu(h, next(it), next(it))
    return h


def _full_consts(ps):
    out = []
    for p in ps:
        out += [p['W'], p['b'].reshape(1, -1)]
        if 'g' in p:
            out += [p['g'].reshape(1, -1), p['be'].reshape(1, -1)]
    return out


def _full_chain(h, it, nl):
    for k in range(nl):
        h = h @ next(it) + next(it)
        if k < nl - 1:
            h = _ln_relu(h, next(it), next(it))
    return h


def _run_mlp(x, ps, br):
    nl = len(ps)

    def body(i, xs, cs):
        return [_full_chain(xs[0], iter(cs), nl)]

    return _pcall(body, [x], _full_consts(ps),
                  [('b', ps[-1]['W'].shape[1])], br)[0]


_NW = 32
_CHUNK = 128


def _sc_mesh():
    return plsc.VectorSubcoreMesh(core_axis_name="c", subcore_axis_name="s")


def _sc_gather_sum2(ta1, tb1, ia1, ib1, ep1, ta2, tb2, ia2, ib2, ep2):
    dw = ta1.shape[1]
    c = _CHUNK

    @functools.partial(
        pl.kernel, mesh=_sc_mesh(),
        out_type=[jax.ShapeDtypeStruct((ep1, dw), jnp.float32),
                  jax.ShapeDtypeStruct((ep2, dw), jnp.float32)],
        scratch_types=[
            pltpu.VMEM((2, c), jnp.int32),
            pltpu.VMEM((2, c), jnp.int32),
            pltpu.VMEM((c, dw), jnp.float32),
            pltpu.VMEM((c, dw), jnp.float32),
            pltpu.VMEM((c, dw), jnp.float32),
            pltpu.VMEM((c, dw), jnp.float32),
            pltpu.VMEM((c, dw), jnp.float32),
            pltpu.VMEM((c, dw), jnp.float32),
        ] + [pltpu.SemaphoreType.DMA] * 6)
    def gk(ta1r, tb1r, ia1r, ib1r, ta2r, tb2r, ia2r, ib2r, out1, out2,
           iav, ibv, a0, a1, b0, b1, o0, o1, sa0, sa1, sb0, sb1, w0, w1):
        wid = lax.axis_index("s") * 2 + lax.axis_index("c")
        av = (a0, a1)
        bv = (b0, b1)
        ov = (o0, o1)
        sa = (sa0, sa1)
        sb = (sb0, sb1)
        wv = (w0, w1)

        def phase(ta, tb, ia, ib, out, ep):
            pw = ep // _NW
            nch = pw // c
            npair = nch // 2
            base0 = wid * pw

            def fire(j, p):
                pltpu.sync_copy(ia.at[pl.ds(base0 + j * c, c)], iav.at[p])
                pltpu.sync_copy(ib.at[pl.ds(base0 + j * c, c)], ibv.at[p])
                pltpu.async_copy(ta.at[iav.at[p]], av[p], sa[p])
                pltpu.async_copy(tb.at[ibv.at[p]], bv[p], sb[p])

            def addstore(j, p):
                pltpu.make_async_copy(ta.at[iav.at[p]], av[p], sa[p]).wait()
                pltpu.make_async_copy(tb.at[ibv.at[p]], bv[p], sb[p]).wait()

                def row(r, c2):
                    for c0 in range(0, dw, 16):
                        ov[p][r, pl.ds(c0, 16)] = (
                            av[p][r, pl.ds(c0, 16)] + bv[p][r, pl.ds(c0, 16)])
                    return c2

                lax.fori_loop(0, c, row, 0)
                pltpu.async_copy(ov[p], out.at[pl.ds(base0 + j * c, c)],
                                 wv[p])

            def wait_wb(p):
                pltpu.make_async_copy(
                    ov[p], out.at[pl.ds(base0, c)], wv[p]).wait()

            fire(0, 0)

            def pair(k, carry):
                j0 = 2 * k
                fire(j0 + 1, 1)

                @pl.when(k >= 1)
                def _():
                    wait_wb(0)

                addstore(j0, 0)

                @pl.when(k < npair - 1)
                def _():
                    fire(j0 + 2, 0)

                @pl.when(k >= 1)
                def _():
                    wait_wb(1)

                addstore(j0 + 1, 1)
                return carry

            lax.fori_loop(0, npair, pair, 0)
            wait_wb(0)
            wait_wb(1)

        phase(ta1r, tb1r, ia1r, ib1r, out1, ep1)
        phase(ta2r, tb2r, ia2r, ib2r, out2, ep2)

    return gk(ta1, tb1, ia1, ib1, ta2, tb2, ia2, ib2)


def _sc_scatter_add2(vals1, idx1, nn1, ep1, vals2, idx2, nn2, ep2):
    dv = vals1.shape[1]
    assert dv == 128 and vals2.shape[1] == 128
    cw = 64
    np1 = -(-nn1 // 128) * 128
    np2 = -(-nn2 // 128) * 128
    rs1 = np1 // 16
    rs2 = np2 // 16

    @functools.partial(
        pl.kernel, mesh=_sc_mesh(),
        out_type=[jax.ShapeDtypeStruct((2, np1, dv), jnp.float32),
                  jax.ShapeDtypeStruct((2, np2, dv), jnp.float32)],
        scratch_types=[
            pltpu.VMEM((1, cw), jnp.int32),
            pltpu.VMEM((1, cw), jnp.int32),
            pltpu.VMEM((cw, dv), jnp.float32),
            pltpu.VMEM((cw, dv), jnp.float32),
            pltpu.VMEM_SHARED((np1, dv), jnp.float32),
            pltpu.VMEM_SHARED((np2, dv), jnp.float32),
            pltpu.SemaphoreType.DMA,
            pltpu.SemaphoreType.DMA,
            pltpu.SemaphoreType.DMA,
            pltpu.SemaphoreType.DMA,
        ])
    def sk(v1, i1, v2, i2, z1, z2, out1, out2, iv0, iv1, b0, b1, tab1, tab2,
           si0, si1, sv0, sv1):
        cc = lax.axis_index("c")
        s = lax.axis_index("s")
        pltpu.sync_copy(z1.at[pl.ds(s * rs1, rs1)],
                        tab1.at[pl.ds(s * rs1, rs1)])
        pltpu.sync_copy(z2.at[pl.ds(s * rs2, rs2)],
                        tab2.at[pl.ds(s * rs2, rs2)])
        plsc.subcore_barrier()
        vv = (b0, b1)
        iv = (iv0, iv1)
        si = (si0, si1)
        sv = (sv0, sv1)

        def phase(v_hbm, i_hbm, tab, ep):
            pw = ep // _NW
            nch = pw // cw
            base0 = (cc * 16 + s) * pw

            def fire(j, p):
                base = base0 + j * cw
                pltpu.async_copy(i_hbm.at[pl.ds(base, cw)], iv[p].at[0],
                                 si[p])
                pltpu.async_copy(v_hbm.at[pl.ds(base, cw)], vv[p], sv[p])

            def flush(j, p):
                base = base0 + j * cw
                pltpu.make_async_copy(
                    i_hbm.at[pl.ds(base, cw)], iv[p].at[0], si[p]).wait()
                pltpu.make_async_copy(
                    v_hbm.at[pl.ds(base, cw)], vv[p], sv[p]).wait()
                pltpu.sync_copy(vv[p], tab.at[iv[p].at[0]], add=True)

            fire(0, 0)

            def pair(k, carry):
                j0 = 2 * k
                fire(j0 + 1, 1)
                flush(j0, 0)

                @pl.when(k < nch // 2 - 1)
                def _():
                    fire(j0 + 2, 0)

                flush(j0 + 1, 1)
                return carry

            lax.fori_loop(0, nch // 2, pair, 0)

        phase(v1, i1, tab1, ep1)
        phase(v2, i2, tab2, ep2)
        plsc.subcore_barrier()
        pltpu.sync_copy(tab1.at[pl.ds(s * rs1, rs1)],
                        out1.at[cc, pl.ds(s * rs1, rs1)])
        pltpu.sync_copy(tab2.at[pl.ds(s * rs2, rs2)],
                        out2.at[cc, pl.ds(s * rs2, rs2)])

    o1, o2 = sk(vals1, idx1, vals2, idx2,
                jnp.zeros((np1, dv), jnp.float32),
                jnp.zeros((np2, dv), jnp.float32))
    return o1[:, :nn1], o2[:, :nn2]


def _enc_proj(x, ps, head_ws, g08, br):
    nl = len(ps)
    consts = _full_consts(ps) + [g08]
    for (wx, wg, b) in head_ws:
        consts += [wx, wg, b.reshape(1, -1)]

    def body(i, xs, cs):
        it = iter(cs)
        h = _full_chain(xs[0], it, nl)
        g = next(it)
        outs = [h]
        for _ in head_ws:
            wx, wg, b = next(it), next(it), next(it)
            outs.append(h @ wx + 0.5 * (g[0:1, :] @ wg + b))
        return outs

    return _pcall(body, [x], consts,
                  [('b', ps[-1]['W'].shape[1])] + [('b', 128)] * len(head_ws),
                  br)


def _edge_block1(e0, gsum, ps, n_real, br):
    consts = [ps[0]['W'][0:64]] + _tail_consts(ps)

    def body(i, xs, cs):
        e0b, gb = xs
        it = iter(cs)
        z = e0b @ next(it) + gb
        h = _tail_chain(z, it, 4)
        rows = i * br + lax.broadcasted_iota(jnp.int32, (br, 1), 0)
        h = jnp.where(rows < n_real, h, 0.0)
        s = jnp.pad(jnp.sum(h, 0, keepdims=True), ((0, 7), (0, 0)))
        return [jnp.pad(h, ((0, 0), (0, 64))), s]

    return _pcall(body, [e0, gsum], consts, [('b', 128), ('s', 64)], br)


def _edge_block2(e0, e1, gsum, gt8, ps, dec_ps, w_out, b_out, n_real, br):
    w0 = ps[0]['W']
    consts = ([w0[0:64], w0[64:128], gt8] + _tail_consts(ps)
              + _full_consts(dec_ps) + [w_out, b_out.reshape(1, -1)])
    dout = w_out.shape[1]

    def body(i, xs, cs):
        e0b, e1b, gb = xs
        it = iter(cs)
        z = e0b @ next(it) + e1b[:, :64] @ next(it) + gb + next(it)[0:1, :]
        h = _tail_chain(z, it, 4)
        rows = i * br + lax.broadcasted_iota(jnp.int32, (br, 1), 0)
        e2 = jnp.where(rows < n_real, h, 0.0)
        hd = _full_chain(h, it, 4)
        y = hd @ next(it) + next(it)
        return [jnp.pad(e2, ((0, 0), (0, 64))), y]

    return _pcall(body, [e0, e1, gsum], consts,
                  [('b', 128), ('b', dout)], br)


def _node_block1(x, agg2, ps, g08, head_ws, br):
    w0 = ps[0]['W']
    consts = ([w0[0:64], w0[64:128], w0[128:192], ps[0]['b'].reshape(1, -1),
               g08] + _tail_consts(ps))
    for (wx, wn) in head_ws:
        consts += [wx, wn]

    def body(i, xs, cs):
        xb, a0, a1 = xs
        it = iter(cs)
        wx, wa, wg, b, g = next(it), next(it), next(it), next(it), next(it)
        z = xb @ wx + (a0 + a1)[:, :64] @ wa + (g[0:1, :] @ wg + b)
        h = _tail_chain(z, it, 4)
        s = jnp.pad(jnp.sum(h, 0, keepdims=True), ((0, 7), (0, 0)))
        outs = [h, s]
        for _ in head_ws:
            whx, whn = next(it), next(it)
            outs.append(xb @ whx + h @ whn)
        return outs

    return _pcall(body, [x, agg2[0], agg2[1]], consts,
                  [('b', 64), ('s', 64)] + [('b', 128)] * len(head_ws), br)


def _node_block2(x0, n1, agg2, ps, dec_ps, g08, g18, br):
    w0 = ps[0]['W']
    consts = ([w0[0:64], w0[64:128], w0[128:192], w0[192:256], w0[256:320],
               ps[0]['b'].reshape(1, -1), g08, g18]
              + _tail_consts(ps) + _full_consts(dec_ps))

    def body(i, xs, cs):
        xb, nb, a0, a1 = xs
        it = iter(cs)
        wxa, wxb, wa, wg0, wg1, b, g0f, g1f = (
            next(it), next(it), next(it), next(it), next(it), next(it),
            next(it), next(it))
        z = (xb @ wxa + nb @ wxb + (a0 + a1)[:, :64] @ wa
             + (g0f[0:1, :] @ wg0 + g1f[0:1, :] @ wg1 + b))
        h = _tail_chain(z, it, 4)
        return [_full_chain(h, it, 4)]

    return _pcall(body, [x0, n1, agg2[0], agg2[1]], consts,
                  [('b', 64)], br)


def _global1(g08, s_ett, s_etp, s_nt, s_np, ps, inv_ett, inv_etp,
             inv_nt, inv_np, gterm_ws):
    w0 = ps[0]['W']
    consts = ([s_ett, s_etp, s_nt, s_np,
               w0[0:64], w0[64:128], w0[128:192], w0[192:256], w0[256:320],
               ps[0]['b'].reshape(1, -1)] + _tail_consts(ps))
    for (wg0, wg1, b) in gterm_ws:
        consts += [wg0, wg1, b.reshape(1, -1)]

    def body(i, xs, cs):
        gb, = xs
        it = iter(cs)
        se, sp, sn, sq = next(it), next(it), next(it), next(it)
        wg, w1, w2, w3, w4, b = (next(it), next(it), next(it), next(it),
                                 next(it), next(it))
        z = (gb @ wg + (se[0:1, :] * inv_ett) @ w1
             + (sp[0:1, :] * inv_etp) @ w2 + (sn[0:1, :] * inv_nt) @ w3
             + (sq[0:1, :] * inv_np) @ w4 + b)
        g1 = _tail_chain(z, it, 4)
        g1r = g1[0:1, :]
        g0r = gb[0:1, :]
        outs = [g1]
        for _ in gterm_ws:
            wg0, wg1_, bb = next(it), next(it), next(it)
            outs.append(jnp.broadcast_to(g0r @ wg0 + g1r @ wg1_ + bb,
                                         (8, 128)))
        return outs

    return _pcall(body, [g08], consts,
                  [('b', 64)] + [('b', 128)] * len(gterm_ws), 8)


def _pad_rows(x, n):
    if x.shape[0] == n:
        return x
    return jnp.pad(x, ((0, n - x.shape[0]),) + ((0, 0),) * (x.ndim - 1))


def kernel(tracks_x, pvs_x, tt_edge_attr, tp_edge_attr, globals_x, params,
           tt_edge_index, tp_edge_index):
    P = params
    nt, npv = tracks_x.shape[0], pvs_x.shape[0]
    ett_n, etp_n = tt_edge_attr.shape[0], tp_edge_attr.shape[0]
    unit = _NW * _CHUNK
    ep_tt = -(-ett_n // unit) * unit
    ep_tp = -(-etp_n // unit) * unit

    tta = _pad_rows(tt_edge_attr, ep_tt)
    tpa = _pad_rows(tp_edge_attr, ep_tp)
    tts = _pad_rows(tt_edge_index[0], ep_tt)
    ttd = _pad_rows(tt_edge_index[1], ep_tt)
    tps = _pad_rows(tp_edge_index[0], ep_tp)
    tpd = _pad_rows(tp_edge_index[1], ep_tp)
    g_in = jnp.pad(globals_x, ((0, 7), (0, 0)))

    brn, bre = 1000, 1024

    b1 = P['block1']
    b2 = P['block2']
    w1tt, b1tt = b1['edge_tt'][0]['W'], b1['edge_tt'][0]['b']
    w1tp, b1tp = b1['edge_tp'][0]['W'], b1['edge_tp'][0]['b']
    w2tt, b2tt = b2['edge_tt'][0]['W'], b2['edge_tt'][0]['b']
    w2tp, b2tp = b2['edge_tp'][0]['W'], b2['edge_tp'][0]['b']

    g08 = _run_mlp(g_in, P['enc_global'], 8)
    xt0, ps_tt, pd_tt, ps_tp = _enc_proj(
        tracks_x, P['enc_node_tracks'],
        [(w1tt[64:128], w1tt[192:256], b1tt),
         (w1tt[128:192], w1tt[192:256], b1tt),
         (w1tp[64:128], w1tp[192:256], b1tp)], g08, brn)
    xp0, pd_tp = _enc_proj(
        pvs_x, P['enc_node_pvs'],
        [(w1tp[128:192], w1tp[192:256], b1tp)], g08, brn)
    ett0 = _run_mlp(tta, P['enc_edge_tt'], bre)
    etp0 = _run_mlp(tpa, P['enc_edge_tp'], bre)

    g1_tt, g1_tp = _sc_gather_sum2(ps_tt, pd_tt, tts, ttd, ep_tt,
                                   ps_tp, pd_tp, tps, tpd, ep_tp)

    e_tt1, s_ett1 = _edge_block1(ett0, g1_tt, b1['edge_tt'], ett_n, bre)
    e_tp1, s_etp1 = _edge_block1(etp0, g1_tp, b1['edge_tp'], etp_n, bre)

    agg_t1, agg_p1 = _sc_scatter_add2(e_tt1, ttd, nt, ep_tt,
                                      e_tp1, tpd, npv, ep_tp)

    n_t1, s_nt1, qs_tt, qd_tt, qs_tp = _node_block1(
        xt0, agg_t1, b1['node_tracks'], g08,
        [(w2tt[128:192], w2tt[192:256]),
         (w2tt[256:320], w2tt[320:384]),
         (w2tp[128:192], w2tp[192:256])], brn)
    n_p1, s_np1, qd_tp = _node_block1(
        xp0, agg_p1, b1['node_pvs'], g08,
        [(w2tp[256:320], w2tp[320:384])], brn)

    g18, gt_tt, gt_tp = _global1(
        g08, s_ett1, s_etp1, s_nt1, s_np1, b1['global'],
        1.0 / ett_n, 1.0 / etp_n, 1.0 / nt, 1.0 / npv,
        [(w2tt[384:448], w2tt[448:512], b2tt),
         (w2tp[384:448], w2tp[448:512], b2tp)])

    g2_tt, g2_tp = _sc_gather_sum2(qs_tt, qd_tt, tts, ttd, ep_tt,
                                   qs_tp, qd_tp, tps, tpd, ep_tp)

    e_tt2, o_tt = _edge_block2(ett0, e_tt1, g2_tt, gt_tt, b2['edge_tt'],
                               P['dec_edge_tt'], P['out_tt']['W'],
                               P['out_tt']['b'], ett_n, bre)
    e_tp2, o_tp = _edge_block2(etp0, e_tp1, g2_tp, gt_tp, b2['edge_tp'],
                               P['dec_edge_tp'], P['out_tp']['W'],
                               P['out_tp']['b'], etp_n, bre)

    agg_t2, agg_p2 = _sc_scatter_add2(e_tt2, ttd, nt, ep_tt,
                                      e_tp2, tpd, npv, ep_tp)

    dxt = _node_block2(xt0, n_t1, agg_t2, b2['node_tracks'],
                       P['dec_node_tracks'], g08, g18, brn)[0]
    dxp = _node_block2(xp0, n_p1, agg_p2, b2['node_pvs'],
                       P['dec_node_pvs'], g08, g18, brn)[0]

    return (o_tt[:ett_n], o_tp[:etp_n], dxt, dxp)
